# baseline (speedup 1.0000x reference)
import functools

import jax
import jax.numpy as jnp
from jax.experimental import pallas as pl
from jax.experimental.pallas import tpu as pltpu

N_NODES = 100000
N_EDGES = 3200000
G = 16
BLOCK_E = 64000
BG = BLOCK_E // G
BLOCK_N = 10000


def _leaky(t):
    return jnp.where(t > 0, t, 0.01 * t)


def _xr_body(x_ref, wr_ref, br_ref, o_ref):
    o_ref[...] = jnp.dot(x_ref[...], wr_ref[...],
                         preferred_element_type=jnp.float32) + br_ref[...]


def _mlp_body(xi_ref, h_ref, dy_ref, w1t_ref, w1b_ref, w2_ref, w3_ref,
              w4_ref, w5_ref, b1_ref, b2_ref, b3_ref, b4_ref, b5_ref,
              rep_ref, o_ref):
    dot = functools.partial(jnp.dot, preferred_element_type=jnp.float32)
    a = _leaky(dot(xi_ref[...], w1t_ref[...]) +
               dot(h_ref[...], w1b_ref[...]) + b1_ref[...])
    a = _leaky(dot(a, w2_ref[...]) + b2_ref[...])
    a = _leaky(dot(a, w3_ref[...]) + b3_ref[...])
    a = _leaky(dot(a, w4_ref[...]) + b4_ref[...])
    m = dot(a, w5_ref[...]) + b5_ref[...]
    dyrep = dot(dy_ref[...], rep_ref[...])
    o_ref[...] = m * dyrep


def _combine_body(s_ref, c_ref, o_ref):
    cnt = jnp.maximum(c_ref[...], 1.0)
    o_ref[...] = s_ref[...] / cnt


def _full(shape):
    return pl.BlockSpec(shape, lambda i: tuple(0 for _ in shape))


def kernel(x, edge_index, h_mat_edge, delta_y, Wr, br, W1, b1, W2, b2,
           W3, b3, W4, b4, W5, b5):
    dst = edge_index[1]

    xr = pl.pallas_call(
        _xr_body,
        grid=(N_NODES // BLOCK_N,),
        in_specs=[pl.BlockSpec((BLOCK_N, 16), lambda i: (i, 0)),
                  _full((16, 4)), _full((1, 4))],
        out_specs=pl.BlockSpec((BLOCK_N, 4), lambda i: (i, 0)),
        out_shape=jax.ShapeDtypeStruct((N_NODES, 4), jnp.float32),
    )(x, Wr, br.reshape(1, 4))

    xi = jnp.take(xr, dst, axis=0)

    eye = jnp.eye(G, dtype=jnp.float32)
    w1t = jnp.kron(eye, W1[:4])
    w1b = jnp.kron(eye, W1[4:])
    w2 = jnp.kron(eye, W2)
    w3 = jnp.kron(eye, W3)
    w4 = jnp.kron(eye, W4)
    w5 = jnp.kron(eye, W5)
    b1t = jnp.tile(b1, G).reshape(1, 256)
    b2t = jnp.tile(b2, G).reshape(1, 256)
    b3t = jnp.tile(b3, G).reshape(1, 256)
    b4t = jnp.tile(b4, G).reshape(1, 256)
    b5t = jnp.tile(b5, G).reshape(1, 64)
    rep = jnp.kron(eye, jnp.ones((1, 4), jnp.float32))

    xi_g = xi.reshape(N_EDGES // G, 64)
    h_g = h_mat_edge.reshape(N_EDGES // G, 64)
    dy_g = delta_y.reshape(N_EDGES // G, 16)

    m_g = pl.pallas_call(
        _mlp_body,
        grid=(N_EDGES // BLOCK_E,),
        in_specs=[pl.BlockSpec((BG, 64), lambda i: (i, 0)),
                  pl.BlockSpec((BG, 64), lambda i: (i, 0)),
                  pl.BlockSpec((BG, 16), lambda i: (i, 0)),
                  _full((64, 256)), _full((64, 256)),
                  _full((256, 256)), _full((256, 256)), _full((256, 256)),
                  _full((256, 64)),
                  _full((1, 256)), _full((1, 256)), _full((1, 256)),
                  _full((1, 256)), _full((1, 64)), _full((16, 64))],
        out_specs=pl.BlockSpec((BG, 64), lambda i: (i, 0)),
        out_shape=jax.ShapeDtypeStruct((N_EDGES // G, 64), jnp.float32),
    )(xi_g, h_g, dy_g, w1t, w1b, w2, w3, w4, w5,
      b1t, b2t, b3t, b4t, b5t, rep)

    m = m_g.reshape(N_EDGES, 4)

    summ = jax.ops.segment_sum(m, dst, num_segments=N_NODES)
    cnt = jax.ops.segment_sum(jnp.ones((N_EDGES, 1), jnp.float32), dst,
                              num_segments=N_NODES)

    out = pl.pallas_call(
        _combine_body,
        grid=(N_NODES // BLOCK_N,),
        in_specs=[pl.BlockSpec((BLOCK_N, 4), lambda i: (i, 0)),
                  pl.BlockSpec((BLOCK_N, 1), lambda i: (i, 0))],
        out_specs=pl.BlockSpec((BLOCK_N, 4), lambda i: (i, 0)),
        out_shape=jax.ShapeDtypeStruct((N_NODES, 4), jnp.float32),
    )(summ, cnt)
    return out
